# baseline (speedup 1.0000x reference)
import functools

import jax
import jax.numpy as jnp
from jax import lax
from jax.experimental import pallas as pl
from jax.experimental.pallas import tpu as pltpu
from jax.experimental.pallas import tpu_sc as plsc

D = 32
INPUT_DIM = 128
NUM_TABLES = 4


def _sc_gather(n_rows, chunk):
    info = plsc.get_sparse_core_info()
    nc, ns = info.num_cores, info.num_subcores
    nw = nc * ns
    rows_per_w = n_rows // nw
    assert n_rows % nw == 0 and rows_per_w % (2 * chunk) == 0
    n_chunks = rows_per_w // chunk

    mesh = plsc.VectorSubcoreMesh(core_axis_name="c", subcore_axis_name="s")

    @functools.partial(
        pl.kernel,
        mesh=mesh,
        compiler_params=pltpu.CompilerParams(use_tc_tiling_on_sc=False),
        out_type=[jax.ShapeDtypeStruct((n_rows, D), jnp.float32)
                  for _ in range(NUM_TABLES)],
        scratch_types=(
            [pltpu.VMEM((rows_per_w,), jnp.int32) for _ in range(NUM_TABLES)]
            + [pltpu.VMEM((chunk, D), jnp.float32)
               for _ in range(2 * NUM_TABLES)]
            + [pltpu.SemaphoreType.DMA for _ in range(5)]
        ),
    )
    def k(tab0, tab1, tab2, tab3, idx0, idx1, idx2, idx3,
          out0, out1, out2, out3,
          iv0, iv1, iv2, iv3,
          r00, r10, r20, r30, r01, r11, r21, r31,
          sem_l, sem_g0, sem_g1, sem_w0, sem_w1):
        wid = lax.axis_index("s") * nc + lax.axis_index("c")
        wbase = wid * rows_per_w
        tabs = (tab0, tab1, tab2, tab3)
        idxs = (idx0, idx1, idx2, idx3)
        outs = (out0, out1, out2, out3)
        idx_v = (iv0, iv1, iv2, iv3)
        rows_v = ((r00, r10, r20, r30), (r01, r11, r21, r31))
        sem_g = (sem_g0, sem_g1)
        sem_w = (sem_w0, sem_w1)

        for t in range(NUM_TABLES):
            pltpu.async_copy(idxs[t].at[pl.ds(wbase, rows_per_w)],
                             idx_v[t], sem_l)
        for t in range(NUM_TABLES):
            pltpu.make_async_copy(idxs[t].at[pl.ds(wbase, rows_per_w)],
                                  idx_v[t], sem_l).wait()

        def fire_gathers(c, p):
            off = c * chunk
            for t in range(NUM_TABLES):
                pltpu.async_copy(
                    tabs[t].at[idx_v[t].at[pl.ds(off, chunk)]],
                    rows_v[p][t], sem_g[p])

        def drain_gathers(p):
            for t in range(NUM_TABLES):
                pltpu.make_async_copy(
                    tabs[t].at[idx_v[t].at[pl.ds(0, chunk)]],
                    rows_v[p][t], sem_g[p]).wait()

        def fire_wb(c, p):
            off = wbase + c * chunk
            for t in range(NUM_TABLES):
                pltpu.async_copy(rows_v[p][t],
                                 outs[t].at[pl.ds(off, chunk)], sem_w[p])

        def drain_wb(p):
            for t in range(NUM_TABLES):
                pltpu.make_async_copy(
                    rows_v[p][t], outs[t].at[pl.ds(0, chunk)],
                    sem_w[p]).wait()

        def process(c, p, is_first_pair):
            if not is_first_pair:
                drain_wb(p)
            fire_gathers(c, p)
            if p == 1:
                drain_gathers(0)
                fire_wb(c - 1, 0)
            elif not is_first_pair:
                drain_gathers(1)
                fire_wb(c - 1, 1)

        process(0, 0, True)
        process(1, 1, True)

        def body(j, _):
            c = 2 * j
            process(c, 0, False)
            process(c + 1, 1, False)
            return ()

        lax.fori_loop(1, n_chunks // 2, body, ())

        drain_gathers(1)
        fire_wb(n_chunks - 1, 1)
        drain_wb(0)
        drain_wb(1)

    return k


def _tc_project(gi, gu, gt, gn, W, b, block_n):
    n_rows = gi.shape[0]
    grid = (n_rows // block_n,)

    def body(gi_ref, gu_ref, gt_ref, gn_ref, w_ref, b_ref, o_ref):
        w = w_ref[...]
        acc = jax.lax.dot_general(
            gi_ref[...], w[0:D, :], (((1,), (0,)), ((), ())),
            preferred_element_type=jnp.float32)
        acc += jax.lax.dot_general(
            gu_ref[...], w[D:2 * D, :], (((1,), (0,)), ((), ())),
            preferred_element_type=jnp.float32)
        acc += jax.lax.dot_general(
            gt_ref[...], w[2 * D:3 * D, :], (((1,), (0,)), ((), ())),
            preferred_element_type=jnp.float32)
        acc += jax.lax.dot_general(
            gn_ref[...], w[3 * D:4 * D, :], (((1,), (0,)), ((), ())),
            preferred_element_type=jnp.float32)
        o_ref[...] = acc + b_ref[...]

    in_block = pl.BlockSpec((block_n, D), lambda i: (i, 0))
    return pl.pallas_call(
        body,
        grid=grid,
        in_specs=[in_block, in_block, in_block, in_block,
                  pl.BlockSpec((4 * D, INPUT_DIM), lambda i: (0, 0)),
                  pl.BlockSpec((1, INPUT_DIM), lambda i: (0, 0))],
        out_specs=pl.BlockSpec((block_n, INPUT_DIM), lambda i: (i, 0)),
        out_shape=jax.ShapeDtypeStruct((n_rows, INPUT_DIM), jnp.float32),
    )(gi, gu, gt, gn, W, b.reshape(1, INPUT_DIM))


def kernel(item, user, tag, interaction, emb_item, emb_user, emb_tag,
           emb_interaction, W, b):
    B, L = item.shape
    n_rows = B * L

    idx_item = item.reshape(-1).astype(jnp.int32)
    idx_user = user.reshape(-1).astype(jnp.int32)
    idx_tag = tag.reshape(-1).astype(jnp.int32)
    idx_int = interaction.reshape(-1).astype(jnp.int32)

    gather = _sc_gather(n_rows, 400)
    gi, gu, gt, gn = gather(emb_item, emb_user, emb_tag, emb_interaction,
                            idx_item, idx_user, idx_tag, idx_int)

    out = _tc_project(gi, gu, gt, gn, W, b, block_n=2048)
    return out.reshape(B, L, INPUT_DIM)


# speedup vs baseline: 2.8050x; 2.8050x over previous
import functools

import jax
import jax.numpy as jnp
from jax import lax
from jax.experimental import pallas as pl
from jax.experimental.pallas import tpu as pltpu
from jax.experimental.pallas import tpu_sc as plsc

D = 32
INPUT_DIM = 128
N_GATHER = 3


def _sc_gather(n_rows, chunk):
    info = plsc.get_sparse_core_info()
    nc, ns = info.num_cores, info.num_subcores
    nw = nc * ns
    rows_per_w = n_rows // nw
    assert n_rows % nw == 0 and rows_per_w % (2 * chunk) == 0
    n_chunks = rows_per_w // chunk

    mesh = plsc.VectorSubcoreMesh(core_axis_name="c", subcore_axis_name="s")

    @functools.partial(
        pl.kernel,
        mesh=mesh,
        compiler_params=pltpu.CompilerParams(use_tc_tiling_on_sc=False),
        out_type=[jax.ShapeDtypeStruct((n_rows, D), jnp.float32)
                  for _ in range(N_GATHER)],
        scratch_types=(
            [pltpu.VMEM((rows_per_w,), jnp.int32) for _ in range(N_GATHER)]
            + [pltpu.VMEM((chunk, D), jnp.float32)
               for _ in range(2 * N_GATHER)]
            + [pltpu.SemaphoreType.DMA for _ in range(5)]
        ),
    )
    def k(tab0, tab1, tab2, idx0, idx1, idx2,
          out0, out1, out2,
          iv0, iv1, iv2,
          r00, r10, r20, r01, r11, r21,
          sem_l, sem_g0, sem_g1, sem_w0, sem_w1):
        wid = lax.axis_index("s") * nc + lax.axis_index("c")
        wbase = wid * rows_per_w
        tabs = (tab0, tab1, tab2)
        idxs = (idx0, idx1, idx2)
        outs = (out0, out1, out2)
        idx_v = (iv0, iv1, iv2)
        rows_v = ((r00, r10, r20), (r01, r11, r21))
        sem_g = (sem_g0, sem_g1)
        sem_w = (sem_w0, sem_w1)

        for t in range(N_GATHER):
            pltpu.async_copy(idxs[t].at[pl.ds(wbase, rows_per_w)],
                             idx_v[t], sem_l)
        for t in range(N_GATHER):
            pltpu.make_async_copy(idxs[t].at[pl.ds(wbase, rows_per_w)],
                                  idx_v[t], sem_l).wait()

        def fire_gathers(c, p):
            off = c * chunk
            for t in range(N_GATHER):
                pltpu.async_copy(
                    tabs[t].at[idx_v[t].at[pl.ds(off, chunk)]],
                    rows_v[p][t], sem_g[p])

        def drain_gathers(p):
            for t in range(N_GATHER):
                pltpu.make_async_copy(
                    tabs[t].at[idx_v[t].at[pl.ds(0, chunk)]],
                    rows_v[p][t], sem_g[p]).wait()

        def fire_wb(c, p):
            off = wbase + c * chunk
            for t in range(N_GATHER):
                pltpu.async_copy(rows_v[p][t],
                                 outs[t].at[pl.ds(off, chunk)], sem_w[p])

        def drain_wb(p):
            for t in range(N_GATHER):
                pltpu.make_async_copy(
                    rows_v[p][t], outs[t].at[pl.ds(0, chunk)],
                    sem_w[p]).wait()

        def process(c, p, is_first_pair):
            if not is_first_pair:
                drain_wb(p)
            fire_gathers(c, p)
            if p == 1:
                drain_gathers(0)
                fire_wb(c - 1, 0)
            elif not is_first_pair:
                drain_gathers(1)
                fire_wb(c - 1, 1)

        process(0, 0, True)
        process(1, 1, True)

        def body(j, _):
            c = 2 * j
            process(c, 0, False)
            process(c + 1, 1, False)
            return ()

        lax.fori_loop(1, n_chunks // 2, body, ())

        drain_gathers(1)
        fire_wb(n_chunks - 1, 1)
        drain_wb(0)
        drain_wb(1)

    return k


def _tc_pint(emb_interaction, W):
    n_int = emb_interaction.shape[0]

    def body(e_ref, w_ref, o_ref):
        o_ref[...] = jnp.zeros_like(o_ref)
        o_ref[0:n_int, :] = jax.lax.dot_general(
            e_ref[...], w_ref[...], (((1,), (0,)), ((), ())),
            preferred_element_type=jnp.float32)

    return pl.pallas_call(
        body,
        out_shape=jax.ShapeDtypeStruct((8, INPUT_DIM), jnp.float32),
    )(emb_interaction, W[3 * D:4 * D, :])


def _tc_project(gi, gu, gt, inter3, p_int, W, b, block_n):
    n_rows = gi.shape[0]
    grid = (n_rows // block_n,)

    def body(gi_ref, gu_ref, gt_ref, int_ref, p_ref, w_ref, b_ref, o_ref):
        w = w_ref[...]
        acc = jax.lax.dot_general(
            gi_ref[...], w[0:D, :], (((1,), (0,)), ((), ())),
            preferred_element_type=jnp.float32)
        acc += jax.lax.dot_general(
            gu_ref[...], w[D:2 * D, :], (((1,), (0,)), ((), ())),
            preferred_element_type=jnp.float32)
        acc += jax.lax.dot_general(
            gt_ref[...], w[2 * D:3 * D, :], (((1,), (0,)), ((), ())),
            preferred_element_type=jnp.float32)
        ids = int_ref[0, 0, :]
        onehot = (ids[:, None]
                  == jax.lax.broadcasted_iota(jnp.int32, (block_n, 8), 1)
                  ).astype(jnp.float32)
        acc += jax.lax.dot_general(
            onehot, p_ref[...], (((1,), (0,)), ((), ())),
            preferred_element_type=jnp.float32)
        o_ref[...] = acc + b_ref[...]

    in_block = pl.BlockSpec((block_n, D), lambda i: (i, 0))
    return pl.pallas_call(
        body,
        grid=grid,
        in_specs=[in_block, in_block, in_block,
                  pl.BlockSpec((1, 1, block_n), lambda i: (i, 0, 0)),
                  pl.BlockSpec((8, INPUT_DIM), lambda i: (0, 0)),
                  pl.BlockSpec((4 * D, INPUT_DIM), lambda i: (0, 0)),
                  pl.BlockSpec((1, INPUT_DIM), lambda i: (0, 0))],
        out_specs=pl.BlockSpec((block_n, INPUT_DIM), lambda i: (i, 0)),
        out_shape=jax.ShapeDtypeStruct((n_rows, INPUT_DIM), jnp.float32),
    )(gi, gu, gt, inter3, p_int, W, b.reshape(1, INPUT_DIM))


def kernel(item, user, tag, interaction, emb_item, emb_user, emb_tag,
           emb_interaction, W, b):
    B, L = item.shape
    n_rows = B * L
    block_n = 2048

    idx_item = item.reshape(-1).astype(jnp.int32)
    idx_user = user.reshape(-1).astype(jnp.int32)
    idx_tag = tag.reshape(-1).astype(jnp.int32)
    inter3 = interaction.reshape(n_rows // block_n, 1, block_n).astype(
        jnp.int32)

    gather = _sc_gather(n_rows, 400)
    gi, gu, gt = gather(emb_item, emb_user, emb_tag,
                        idx_item, idx_user, idx_tag)

    p_int = _tc_pint(emb_interaction, W)
    out = _tc_project(gi, gu, gt, inter3, p_int, W, b, block_n)
    return out.reshape(B, L, INPUT_DIM)


# speedup vs baseline: 3.8565x; 1.3749x over previous
import functools

import jax
import jax.numpy as jnp
from jax import lax
from jax.experimental import pallas as pl
from jax.experimental.pallas import tpu as pltpu
from jax.experimental.pallas import tpu_sc as plsc

D = 32
INPUT_DIM = 128
N_GATHER = 3


def _sc_gather(n_rows, chunk):
    info = plsc.get_sparse_core_info()
    nc, ns = info.num_cores, info.num_subcores
    nw = nc * ns
    rows_per_w = n_rows // nw
    assert n_rows % nw == 0 and rows_per_w % (2 * chunk) == 0
    n_chunks = rows_per_w // chunk

    mesh = plsc.VectorSubcoreMesh(core_axis_name="c", subcore_axis_name="s")

    @functools.partial(
        pl.kernel,
        mesh=mesh,
        compiler_params=pltpu.CompilerParams(use_tc_tiling_on_sc=False),
        out_type=[jax.ShapeDtypeStruct((n_rows, D), jnp.float32)
                  for _ in range(N_GATHER)],
        scratch_types=(
            [pltpu.VMEM((rows_per_w,), jnp.int32) for _ in range(N_GATHER)]
            + [pltpu.VMEM((chunk, D), jnp.float32)
               for _ in range(2 * N_GATHER)]
            + [pltpu.SemaphoreType.DMA for _ in range(5)]
        ),
    )
    def k(tab0, tab1, tab2, idx0, idx1, idx2,
          out0, out1, out2,
          iv0, iv1, iv2,
          r00, r10, r20, r01, r11, r21,
          sem_l, sem_g0, sem_g1, sem_w0, sem_w1):
        wid = lax.axis_index("s") * nc + lax.axis_index("c")
        wbase = wid * rows_per_w
        tabs = (tab0, tab1, tab2)
        idxs = (idx0, idx1, idx2)
        outs = (out0, out1, out2)
        idx_v = (iv0, iv1, iv2)
        rows_v = ((r00, r10, r20), (r01, r11, r21))
        sem_g = (sem_g0, sem_g1)
        sem_w = (sem_w0, sem_w1)

        for t in range(N_GATHER):
            pltpu.async_copy(idxs[t].at[pl.ds(wbase, rows_per_w)],
                             idx_v[t], sem_l)
        for t in range(N_GATHER):
            pltpu.make_async_copy(idxs[t].at[pl.ds(wbase, rows_per_w)],
                                  idx_v[t], sem_l).wait()

        def fire_gathers(c, p):
            off = c * chunk
            for t in range(N_GATHER):
                pltpu.async_copy(
                    tabs[t].at[idx_v[t].at[pl.ds(off, chunk)]],
                    rows_v[p][t], sem_g[p])

        def drain_gathers(p):
            for t in range(N_GATHER):
                pltpu.make_async_copy(
                    tabs[t].at[idx_v[t].at[pl.ds(0, chunk)]],
                    rows_v[p][t], sem_g[p]).wait()

        def fire_wb(c, p):
            off = wbase + c * chunk
            for t in range(N_GATHER):
                pltpu.async_copy(rows_v[p][t],
                                 outs[t].at[pl.ds(off, chunk)], sem_w[p])

        def drain_wb(p):
            for t in range(N_GATHER):
                pltpu.make_async_copy(
                    rows_v[p][t], outs[t].at[pl.ds(0, chunk)],
                    sem_w[p]).wait()

        def process(c, p, is_first_pair):
            if not is_first_pair:
                drain_wb(p)
            fire_gathers(c, p)
            if p == 1:
                drain_gathers(0)
                fire_wb(c - 1, 0)
            elif not is_first_pair:
                drain_gathers(1)
                fire_wb(c - 1, 1)

        process(0, 0, True)
        process(1, 1, True)

        def body(j, _):
            c = 2 * j
            process(c, 0, False)
            process(c + 1, 1, False)
            return ()

        lax.fori_loop(1, n_chunks // 2, body, ())

        drain_gathers(1)
        fire_wb(n_chunks - 1, 1)
        drain_wb(0)
        drain_wb(1)

    return k


def _tc_pint(emb_interaction, W):
    n_int = emb_interaction.shape[0]

    def body(e_ref, w_ref, o_ref):
        o_ref[...] = jnp.zeros_like(o_ref)
        o_ref[0:n_int, :] = jax.lax.dot_general(
            e_ref[...], w_ref[...], (((1,), (0,)), ((), ())),
            preferred_element_type=jnp.float32)

    return pl.pallas_call(
        body,
        out_shape=jax.ShapeDtypeStruct((8, INPUT_DIM), jnp.float32),
    )(emb_interaction, W[3 * D:4 * D, :])


def _tc_project_packed(gi_p, gu_p, gt_p, int_j, wt, p_int_j, b_tile,
                       block_p):
    n_pack = gi_p.shape[0]
    grid = (n_pack // block_p,)

    def body(gi_ref, gu_ref, gt_ref, i0, i1, i2, i3,
             w1, w2, w3, p0, p1, p2, p3, b_ref, o_ref):
        dn = (((1,), (0,)), ((), ()))
        acc = jax.lax.dot_general(
            gi_ref[...].astype(jnp.bfloat16), w1[...], dn,
            preferred_element_type=jnp.float32)
        acc += jax.lax.dot_general(
            gu_ref[...].astype(jnp.bfloat16), w2[...], dn,
            preferred_element_type=jnp.float32)
        acc += jax.lax.dot_general(
            gt_ref[...].astype(jnp.bfloat16), w3[...], dn,
            preferred_element_type=jnp.float32)
        for idr, pr in ((i0, p0), (i1, p1), (i2, p2), (i3, p3)):
            ids = idr[0, 0, :]
            onehot = (ids[:, None]
                      == jax.lax.broadcasted_iota(jnp.int32, (block_p, 8), 1)
                      ).astype(jnp.float32)
            acc += jax.lax.dot_general(onehot, pr[...], dn,
                                       preferred_element_type=jnp.float32)
        o_ref[...] = acc + b_ref[...]

    in_block = pl.BlockSpec((block_p, 4 * D), lambda i: (i, 0))
    idx_block = pl.BlockSpec((1, 1, block_p), lambda i: (i, 0, 0))
    w_block = pl.BlockSpec((4 * D, 4 * INPUT_DIM), lambda i: (0, 0))
    p_block = pl.BlockSpec((8, 4 * INPUT_DIM), lambda i: (0, 0))
    return pl.pallas_call(
        body,
        grid=grid,
        in_specs=([in_block, in_block, in_block]
                  + [idx_block] * 4 + [w_block] * 3 + [p_block] * 4
                  + [pl.BlockSpec((1, 4 * INPUT_DIM), lambda i: (0, 0))]),
        out_specs=pl.BlockSpec((block_p, 4 * INPUT_DIM), lambda i: (i, 0)),
        out_shape=jax.ShapeDtypeStruct((n_pack, 4 * INPUT_DIM), jnp.float32),
    )(gi_p, gu_p, gt_p, *int_j, *wt, *p_int_j, b_tile)


def kernel(item, user, tag, interaction, emb_item, emb_user, emb_tag,
           emb_interaction, W, b):
    B, L = item.shape
    n_rows = B * L
    n_pack = n_rows // 4
    block_p = 512

    idx_item = item.T.reshape(-1).astype(jnp.int32)
    idx_user = user.T.reshape(-1).astype(jnp.int32)
    idx_tag = tag.T.reshape(-1).astype(jnp.int32)
    inter_lm = interaction.T.reshape(-1).astype(jnp.int32)
    int_j = [inter_lm[j::4].reshape(n_pack // block_p, 1, block_p)
             for j in range(4)]

    gather = _sc_gather(n_rows, 400)
    gi, gu, gt = gather(emb_item, emb_user, emb_tag,
                        idx_item, idx_user, idx_tag)
    gi_p = gi.reshape(n_pack, 4 * D)
    gu_p = gu.reshape(n_pack, 4 * D)
    gt_p = gt.reshape(n_pack, 4 * D)

    eye4 = jnp.eye(4, dtype=jnp.bfloat16)
    wt = [jnp.kron(eye4, W[t * D:(t + 1) * D, :].astype(jnp.bfloat16))
          for t in range(3)]
    p_int = _tc_pint(emb_interaction, W)
    zeros8 = jnp.zeros((8, INPUT_DIM), jnp.float32)
    p_int_j = [jnp.concatenate(
        [p_int if j == k else zeros8 for k in range(4)], axis=1)
        for j in range(4)]
    b_tile = jnp.tile(b, 4).reshape(1, 4 * INPUT_DIM)

    out = _tc_project_packed(gi_p, gu_p, gt_p, int_j, wt, p_int_j, b_tile,
                             block_p)
    return out.reshape(L, B, INPUT_DIM).transpose(1, 0, 2)


# speedup vs baseline: 4.8688x; 1.2625x over previous
import functools

import jax
import jax.numpy as jnp
from jax import lax
from jax.experimental import pallas as pl
from jax.experimental.pallas import tpu as pltpu
from jax.experimental.pallas import tpu_sc as plsc

D = 32
INPUT_DIM = 128
N_GATHER = 3


def _sc_gather(n_rows, chunk):
    info = plsc.get_sparse_core_info()
    nc, ns = info.num_cores, info.num_subcores
    nw = nc * ns
    rows_per_w = n_rows // nw
    assert n_rows % nw == 0 and rows_per_w % (2 * chunk) == 0
    n_chunks = rows_per_w // chunk

    mesh = plsc.VectorSubcoreMesh(core_axis_name="c", subcore_axis_name="s")

    @functools.partial(
        pl.kernel,
        mesh=mesh,
        compiler_params=pltpu.CompilerParams(use_tc_tiling_on_sc=False),
        out_type=[jax.ShapeDtypeStruct((n_rows, D), jnp.float32)
                  for _ in range(N_GATHER)],
        scratch_types=(
            [pltpu.VMEM((rows_per_w,), jnp.int32) for _ in range(N_GATHER)]
            + [pltpu.VMEM((chunk, D), jnp.float32)
               for _ in range(2 * N_GATHER)]
            + [pltpu.SemaphoreType.DMA for _ in range(5)]
        ),
    )
    def k(tab0, tab1, tab2, idx0, idx1, idx2,
          out0, out1, out2,
          iv0, iv1, iv2,
          r00, r10, r20, r01, r11, r21,
          sem_l, sem_g0, sem_g1, sem_w0, sem_w1):
        wid = lax.axis_index("s") * nc + lax.axis_index("c")
        wbase = wid * rows_per_w
        tabs = (tab0, tab1, tab2)
        idxs = (idx0, idx1, idx2)
        outs = (out0, out1, out2)
        idx_v = (iv0, iv1, iv2)
        rows_v = ((r00, r10, r20), (r01, r11, r21))
        sem_g = (sem_g0, sem_g1)
        sem_w = (sem_w0, sem_w1)

        for t in range(N_GATHER):
            pltpu.async_copy(idxs[t].at[pl.ds(wbase, rows_per_w)],
                             idx_v[t], sem_l)
        for t in range(N_GATHER):
            pltpu.make_async_copy(idxs[t].at[pl.ds(wbase, rows_per_w)],
                                  idx_v[t], sem_l).wait()

        def fire_gathers(c, p):
            off = c * chunk
            for t in range(N_GATHER):
                pltpu.async_copy(
                    tabs[t].at[idx_v[t].at[pl.ds(off, chunk)]],
                    rows_v[p][t], sem_g[p])

        def drain_gathers(p):
            for t in range(N_GATHER):
                pltpu.make_async_copy(
                    tabs[t].at[idx_v[t].at[pl.ds(0, chunk)]],
                    rows_v[p][t], sem_g[p]).wait()

        def fire_wb(c, p):
            off = wbase + c * chunk
            for t in range(N_GATHER):
                pltpu.async_copy(rows_v[p][t],
                                 outs[t].at[pl.ds(off, chunk)], sem_w[p])

        def drain_wb(p):
            for t in range(N_GATHER):
                pltpu.make_async_copy(
                    rows_v[p][t], outs[t].at[pl.ds(0, chunk)],
                    sem_w[p]).wait()

        def process(c, p, is_first_pair):
            if not is_first_pair:
                drain_wb(p)
            fire_gathers(c, p)
            if p == 1:
                drain_gathers(0)
                fire_wb(c - 1, 0)
            elif not is_first_pair:
                drain_gathers(1)
                fire_wb(c - 1, 1)

        process(0, 0, True)
        process(1, 1, True)

        def body(j, _):
            c = 2 * j
            process(c, 0, False)
            process(c + 1, 1, False)
            return ()

        lax.fori_loop(1, n_chunks // 2, body, ())

        drain_gathers(1)
        fire_wb(n_chunks - 1, 1)
        drain_wb(0)
        drain_wb(1)

    return k


def _tc_pint(emb_interaction, W):
    n_int = emb_interaction.shape[0]

    def body(e_ref, w_ref, o_ref):
        o_ref[...] = jnp.zeros_like(o_ref)
        o_ref[0:n_int, :] = jax.lax.dot_general(
            e_ref[...], w_ref[...], (((1,), (0,)), ((), ())),
            preferred_element_type=jnp.float32)

    return pl.pallas_call(
        body,
        out_shape=jax.ShapeDtypeStruct((8, INPUT_DIM), jnp.float32),
    )(emb_interaction, W[3 * D:4 * D, :])


def _tc_pack_table(tab_t):
    n_rows = tab_t.shape[1]
    block_v = 2048
    q = block_v // 4
    grid = (pl.cdiv(n_rows, block_v),)

    def body(t_ref, eye_ref, o_ref):
        xt = jax.lax.dot_general(
            t_ref[...], eye_ref[...], (((0,), (0,)), ((), ())),
            preferred_element_type=jnp.float32)
        for j in range(4):
            o_ref[:, D * j:D * (j + 1)] = xt[q * j:q * (j + 1), :]

    n_pad = grid[0] * block_v
    return pl.pallas_call(
        body,
        grid=grid,
        in_specs=[pl.BlockSpec((D, block_v), lambda i: (0, i)),
                  pl.BlockSpec((D, D), lambda i: (0, 0))],
        out_specs=pl.BlockSpec((q, 4 * D), lambda i: (i, 0)),
        out_shape=jax.ShapeDtypeStruct((n_pad // 4, 4 * D), jnp.float32),
    )(tab_t, jnp.eye(D, dtype=jnp.float32))


def _tc_project_packed(gi_p, gu_p, gt_p, inter3, wt, p_int, b, block_p):
    n_pack = gi_p.shape[0]
    grid = (n_pack // block_p,)
    block_n = 4 * block_p

    def body(gi_ref, gu_ref, gt_ref, int_ref, w1, w2, w3, p_ref, b_ref,
             o_ref):
        dn = (((1,), (0,)), ((), ()))
        acc = jax.lax.dot_general(
            gi_ref[...].astype(jnp.bfloat16), w1[...], dn,
            preferred_element_type=jnp.float32)
        acc += jax.lax.dot_general(
            gu_ref[...].astype(jnp.bfloat16), w2[...], dn,
            preferred_element_type=jnp.float32)
        acc += jax.lax.dot_general(
            gt_ref[...].astype(jnp.bfloat16), w3[...], dn,
            preferred_element_type=jnp.float32)
        ids = int_ref[0, 0, :]
        onehot = (ids[:, None]
                  == jax.lax.broadcasted_iota(jnp.int32, (block_n, 8), 1)
                  ).astype(jnp.float32)
        intc = jax.lax.dot_general(onehot, p_ref[...], dn,
                                   preferred_element_type=jnp.float32)
        o_ref[...] = acc.reshape(block_n, INPUT_DIM) + intc + b_ref[...]

    in_block = pl.BlockSpec((block_p, 4 * D), lambda i: (i, 0))
    return pl.pallas_call(
        body,
        grid=grid,
        in_specs=[in_block, in_block, in_block,
                  pl.BlockSpec((1, 1, block_n), lambda i: (i, 0, 0)),
                  pl.BlockSpec((4 * D, 4 * INPUT_DIM), lambda i: (0, 0)),
                  pl.BlockSpec((4 * D, 4 * INPUT_DIM), lambda i: (0, 0)),
                  pl.BlockSpec((4 * D, 4 * INPUT_DIM), lambda i: (0, 0)),
                  pl.BlockSpec((8, INPUT_DIM), lambda i: (0, 0)),
                  pl.BlockSpec((1, INPUT_DIM), lambda i: (0, 0))],
        out_specs=pl.BlockSpec((block_n, INPUT_DIM), lambda i: (i, 0)),
        out_shape=jax.ShapeDtypeStruct((4 * n_pack, INPUT_DIM), jnp.float32),
    )(gi_p, gu_p, gt_p, inter3, *wt, p_int, b.reshape(1, INPUT_DIM))


def kernel(item, user, tag, interaction, emb_item, emb_user, emb_tag,
           emb_interaction, W, b):
    B, L = item.shape
    n_rows = B * L
    n_pack = n_rows // 4
    block_p = 512

    def perm(v):
        return (v // 2048) * 2048 + 4 * (v % 512) + (v % 2048) // 512

    idx_item = perm(item.T.reshape(-1).astype(jnp.int32))
    idx_user = perm(user.T.reshape(-1).astype(jnp.int32))
    idx_tag = tag.T.reshape(-1).astype(jnp.int32)
    inter3 = interaction.T.reshape(-1).astype(jnp.int32).reshape(
        n_pack // block_p, 1, 4 * block_p)

    item_pk = _tc_pack_table(emb_item.T)
    user_pk = _tc_pack_table(emb_user.T)
    item_rm = item_pk.reshape(item_pk.shape[0] * 4, D)
    user_rm = user_pk.reshape(user_pk.shape[0] * 4, D)

    gather = _sc_gather(n_rows, 400)
    gi, gu, gt = gather(item_rm, user_rm, emb_tag,
                        idx_item, idx_user, idx_tag)
    gi_p = gi.reshape(n_pack, 4 * D)
    gu_p = gu.reshape(n_pack, 4 * D)
    gt_p = gt.reshape(n_pack, 4 * D)

    eye4 = jnp.eye(4, dtype=jnp.bfloat16)
    wt = [jnp.kron(eye4, W[t * D:(t + 1) * D, :].astype(jnp.bfloat16))
          for t in range(3)]
    p_int = _tc_pint(emb_interaction, W)

    out = _tc_project_packed(gi_p, gu_p, gt_p, inter3, wt, p_int, b,
                             block_p)
    return out.reshape(L, B, INPUT_DIM).transpose(1, 0, 2)


# speedup vs baseline: 8.3306x; 1.7110x over previous
import functools

import jax
import jax.numpy as jnp
from jax import lax
from jax.experimental import pallas as pl
from jax.experimental.pallas import tpu as pltpu
from jax.experimental.pallas import tpu_sc as plsc

D = 32
INPUT_DIM = 128
N_GATHER = 3
PACK_BLOCK = 16384


def _sc_gather(n_rows, chunk):
    info = plsc.get_sparse_core_info()
    nc, ns = info.num_cores, info.num_subcores
    nw = nc * ns
    rows_per_w = n_rows // nw
    assert n_rows % nw == 0 and rows_per_w % (2 * chunk) == 0
    n_chunks = rows_per_w // chunk

    mesh = plsc.VectorSubcoreMesh(core_axis_name="c", subcore_axis_name="s")

    @functools.partial(
        pl.kernel,
        mesh=mesh,
        compiler_params=pltpu.CompilerParams(use_tc_tiling_on_sc=False),
        out_type=[jax.ShapeDtypeStruct((n_rows, D), jnp.float32)
                  for _ in range(N_GATHER)],
        scratch_types=(
            [pltpu.VMEM((rows_per_w,), jnp.int32) for _ in range(N_GATHER)]
            + [pltpu.VMEM((chunk, D), jnp.float32)
               for _ in range(2 * N_GATHER)]
            + [pltpu.SemaphoreType.DMA for _ in range(5)]
        ),
    )
    def k(tab0, tab1, tab2, idx0, idx1, idx2,
          out0, out1, out2,
          iv0, iv1, iv2,
          r00, r10, r20, r01, r11, r21,
          sem_l, sem_g0, sem_g1, sem_w0, sem_w1):
        wid = lax.axis_index("s") * nc + lax.axis_index("c")
        wbase = wid * rows_per_w
        tabs = (tab0, tab1, tab2)
        idxs = (idx0, idx1, idx2)
        outs = (out0, out1, out2)
        idx_v = (iv0, iv1, iv2)
        rows_v = ((r00, r10, r20), (r01, r11, r21))
        sem_g = (sem_g0, sem_g1)
        sem_w = (sem_w0, sem_w1)

        for t in range(N_GATHER):
            pltpu.async_copy(idxs[t].at[pl.ds(wbase, rows_per_w)],
                             idx_v[t], sem_l)
        for t in range(N_GATHER):
            pltpu.make_async_copy(idxs[t].at[pl.ds(wbase, rows_per_w)],
                                  idx_v[t], sem_l).wait()

        def fire_gathers(c, p):
            off = c * chunk
            for t in range(N_GATHER):
                pltpu.async_copy(
                    tabs[t].at[idx_v[t].at[pl.ds(off, chunk)]],
                    rows_v[p][t], sem_g[p])

        def drain_gathers(p):
            for t in range(N_GATHER):
                pltpu.make_async_copy(
                    tabs[t].at[idx_v[t].at[pl.ds(0, chunk)]],
                    rows_v[p][t], sem_g[p]).wait()

        def fire_wb(c, p):
            off = wbase + c * chunk
            for t in range(N_GATHER):
                pltpu.async_copy(rows_v[p][t],
                                 outs[t].at[pl.ds(off, chunk)], sem_w[p])

        def drain_wb(p):
            for t in range(N_GATHER):
                pltpu.make_async_copy(
                    rows_v[p][t], outs[t].at[pl.ds(0, chunk)],
                    sem_w[p]).wait()

        def process(c, p, is_first_pair):
            if not is_first_pair:
                drain_wb(p)
            fire_gathers(c, p)
            if p == 1:
                drain_gathers(0)
                fire_wb(c - 1, 0)
            elif not is_first_pair:
                drain_gathers(1)
                fire_wb(c - 1, 1)

        process(0, 0, True)
        process(1, 1, True)

        def body(j, _):
            c = 2 * j
            process(c, 0, False)
            process(c + 1, 1, False)
            return ()

        lax.fori_loop(1, n_chunks // 2, body, ())

        drain_gathers(1)
        fire_wb(n_chunks - 1, 1)
        drain_wb(0)
        drain_wb(1)

    return k


def _tc_pint(emb_interaction, W):
    n_int = emb_interaction.shape[0]

    def body(e_ref, w_ref, o_ref):
        o_ref[...] = jnp.zeros_like(o_ref)
        o_ref[0:n_int, :] = jax.lax.dot_general(
            e_ref[...], w_ref[...], (((1,), (0,)), ((), ())),
            preferred_element_type=jnp.float32)

    return pl.pallas_call(
        body,
        out_shape=jax.ShapeDtypeStruct((8, INPUT_DIM), jnp.float32),
    )(emb_interaction, W[3 * D:4 * D, :])


def _tc_pack_table(tab_t):
    n_rows = tab_t.shape[1]
    block_v = PACK_BLOCK
    q = block_v // 4
    grid = (pl.cdiv(n_rows, block_v),)

    def body(t_ref, e_ref, o_ref):
        i = pl.program_id(0)
        cols = (jax.lax.broadcasted_iota(jnp.int32, (D, block_v), 1)
                + i * block_v)
        x = jnp.where(cols < n_rows, t_ref[...], 0.0)
        acc = jax.lax.dot_general(
            x[:, 0:q], e_ref[0:D, :], (((0,), (0,)), ((), ())),
            preferred_element_type=jnp.float32)
        for j in range(1, 4):
            acc += jax.lax.dot_general(
                x[:, q * j:q * (j + 1)], e_ref[D * j:D * (j + 1), :],
                (((0,), (0,)), ((), ())),
                preferred_element_type=jnp.float32)
        o_ref[...] = acc

    expander = jnp.eye(4 * D, dtype=jnp.float32)

    n_pad = grid[0] * block_v
    return pl.pallas_call(
        body,
        grid=grid,
        in_specs=[pl.BlockSpec((D, block_v), lambda i: (0, i)),
                  pl.BlockSpec((4 * D, 4 * D), lambda i: (0, 0))],
        out_specs=pl.BlockSpec((q, 4 * D), lambda i: (i, 0)),
        out_shape=jax.ShapeDtypeStruct((n_pad // 4, 4 * D), jnp.float32),
    )(tab_t, expander)


def _tc_project_packed(gi_p, gu_p, gt_p, inter3, wt, p_int, b, block_p):
    n_pack = gi_p.shape[0]
    grid = (n_pack // block_p,)
    block_n = 4 * block_p

    def body(gi_ref, gu_ref, gt_ref, int_ref, w1, w2, w3, p_ref, b_ref,
             o_ref):
        dn = (((1,), (0,)), ((), ()))
        acc = jax.lax.dot_general(
            gi_ref[...].astype(jnp.bfloat16), w1[...], dn,
            preferred_element_type=jnp.float32)
        acc += jax.lax.dot_general(
            gu_ref[...].astype(jnp.bfloat16), w2[...], dn,
            preferred_element_type=jnp.float32)
        acc += jax.lax.dot_general(
            gt_ref[...].astype(jnp.bfloat16), w3[...], dn,
            preferred_element_type=jnp.float32)
        ids = int_ref[0, 0, :]
        onehot = (ids[:, None]
                  == jax.lax.broadcasted_iota(jnp.int32, (block_n, 8), 1)
                  ).astype(jnp.float32)
        intc = jax.lax.dot_general(onehot, p_ref[...], dn,
                                   preferred_element_type=jnp.float32)
        o_ref[...] = acc.reshape(block_n, INPUT_DIM) + intc + b_ref[...]

    in_block = pl.BlockSpec((block_p, 4 * D), lambda i: (i, 0))
    return pl.pallas_call(
        body,
        grid=grid,
        in_specs=[in_block, in_block, in_block,
                  pl.BlockSpec((1, 1, block_n), lambda i: (i, 0, 0)),
                  pl.BlockSpec((4 * D, 4 * INPUT_DIM), lambda i: (0, 0)),
                  pl.BlockSpec((4 * D, 4 * INPUT_DIM), lambda i: (0, 0)),
                  pl.BlockSpec((4 * D, 4 * INPUT_DIM), lambda i: (0, 0)),
                  pl.BlockSpec((8, INPUT_DIM), lambda i: (0, 0)),
                  pl.BlockSpec((1, INPUT_DIM), lambda i: (0, 0))],
        out_specs=pl.BlockSpec((block_n, INPUT_DIM), lambda i: (i, 0)),
        out_shape=jax.ShapeDtypeStruct((4 * n_pack, INPUT_DIM), jnp.float32),
    )(gi_p, gu_p, gt_p, inter3, *wt, p_int, b.reshape(1, INPUT_DIM))


def kernel(item, user, tag, interaction, emb_item, emb_user, emb_tag,
           emb_interaction, W, b):
    B, L = item.shape
    n_rows = B * L
    n_pack = n_rows // 4
    block_p = 512

    bv, qv = PACK_BLOCK, PACK_BLOCK // 4

    def perm(v):
        return (v // bv) * bv + 4 * (v % qv) + (v % bv) // qv

    idx_item = perm(item.T.reshape(-1).astype(jnp.int32))
    idx_user = perm(user.T.reshape(-1).astype(jnp.int32))
    idx_tag = tag.T.reshape(-1).astype(jnp.int32)
    inter3 = interaction.T.reshape(-1).astype(jnp.int32).reshape(
        n_pack // block_p, 1, 4 * block_p)

    item_pk = _tc_pack_table(emb_item.T)
    user_pk = _tc_pack_table(emb_user.T)
    item_rm = item_pk.reshape(item_pk.shape[0] * 4, D)
    user_rm = user_pk.reshape(user_pk.shape[0] * 4, D)

    gather = _sc_gather(n_rows, 400)
    gi, gu, gt = gather(item_rm, user_rm, emb_tag,
                        idx_item, idx_user, idx_tag)
    gi_p = gi.reshape(n_pack, 4 * D)
    gu_p = gu.reshape(n_pack, 4 * D)
    gt_p = gt.reshape(n_pack, 4 * D)

    eye4 = jnp.eye(4, dtype=jnp.bfloat16)
    wt = [jnp.kron(eye4, W[t * D:(t + 1) * D, :].astype(jnp.bfloat16))
          for t in range(3)]
    p_int = _tc_pint(emb_interaction, W)

    out = _tc_project_packed(gi_p, gu_p, gt_p, inter3, wt, p_int, b,
                             block_p)
    return out.reshape(L, B, INPUT_DIM).transpose(1, 0, 2)


# speedup vs baseline: 8.9455x; 1.0738x over previous
import functools

import jax
import jax.numpy as jnp
from jax import lax
from jax.experimental import pallas as pl
from jax.experimental.pallas import tpu as pltpu
from jax.experimental.pallas import tpu_sc as plsc

D = 32
INPUT_DIM = 128
N_GATHER = 3
PACK_BLOCK = 16384


def _sc_gather(n_rows, chunk):
    info = plsc.get_sparse_core_info()
    nc, ns = info.num_cores, info.num_subcores
    nw = nc * ns
    rows_per_w = n_rows // nw
    assert n_rows % nw == 0 and rows_per_w % (2 * chunk) == 0
    n_chunks = rows_per_w // chunk

    mesh = plsc.VectorSubcoreMesh(core_axis_name="c", subcore_axis_name="s")

    @functools.partial(
        pl.kernel,
        mesh=mesh,
        compiler_params=pltpu.CompilerParams(use_tc_tiling_on_sc=False),
        out_type=[jax.ShapeDtypeStruct((n_rows, D), jnp.float32)
                  for _ in range(N_GATHER)],
        scratch_types=(
            [pltpu.VMEM((rows_per_w,), jnp.int32) for _ in range(N_GATHER)]
            + [pltpu.VMEM((chunk, D), jnp.float32)
               for _ in range(2 * N_GATHER)]
            + [pltpu.SemaphoreType.DMA for _ in range(5)]
        ),
    )
    def k(tab0, tab1, tab2, idx0, idx1, idx2,
          out0, out1, out2,
          iv0, iv1, iv2,
          r00, r10, r20, r01, r11, r21,
          sem_l, sem_g0, sem_g1, sem_w0, sem_w1):
        wid = lax.axis_index("s") * nc + lax.axis_index("c")
        wbase = wid * rows_per_w
        tabs = (tab0, tab1, tab2)
        idxs = (idx0, idx1, idx2)
        outs = (out0, out1, out2)
        idx_v = (iv0, iv1, iv2)
        rows_v = ((r00, r10, r20), (r01, r11, r21))
        sem_g = (sem_g0, sem_g1)
        sem_w = (sem_w0, sem_w1)

        for t in range(N_GATHER):
            pltpu.async_copy(idxs[t].at[pl.ds(wbase, rows_per_w)],
                             idx_v[t], sem_l)
        for t in range(N_GATHER):
            pltpu.make_async_copy(idxs[t].at[pl.ds(wbase, rows_per_w)],
                                  idx_v[t], sem_l).wait()

        def fire_gathers(c, p):
            off = c * chunk
            for t in range(N_GATHER):
                pltpu.async_copy(
                    tabs[t].at[idx_v[t].at[pl.ds(off, chunk)]],
                    rows_v[p][t], sem_g[p])

        def drain_gathers(p):
            for t in range(N_GATHER):
                pltpu.make_async_copy(
                    tabs[t].at[idx_v[t].at[pl.ds(0, chunk)]],
                    rows_v[p][t], sem_g[p]).wait()

        def fire_wb(c, p):
            off = wbase + c * chunk
            for t in range(N_GATHER):
                pltpu.async_copy(rows_v[p][t],
                                 outs[t].at[pl.ds(off, chunk)], sem_w[p])

        def drain_wb(p):
            for t in range(N_GATHER):
                pltpu.make_async_copy(
                    rows_v[p][t], outs[t].at[pl.ds(0, chunk)],
                    sem_w[p]).wait()

        def process(c, p, is_first_pair):
            if not is_first_pair:
                drain_wb(p)
            fire_gathers(c, p)
            if p == 1:
                drain_gathers(0)
                fire_wb(c - 1, 0)
            elif not is_first_pair:
                drain_gathers(1)
                fire_wb(c - 1, 1)

        process(0, 0, True)
        process(1, 1, True)

        def body(j, _):
            c = 2 * j
            process(c, 0, False)
            process(c + 1, 1, False)
            return ()

        lax.fori_loop(1, n_chunks // 2, body, ())

        drain_gathers(1)
        fire_wb(n_chunks - 1, 1)
        drain_wb(0)
        drain_wb(1)

    return k


def _tc_pint(emb_interaction, W):
    n_int = emb_interaction.shape[0]

    def body(e_ref, w_ref, o_ref):
        o_ref[...] = jnp.zeros_like(o_ref)
        o_ref[0:n_int, :] = jax.lax.dot_general(
            e_ref[...], w_ref[...], (((1,), (0,)), ((), ())),
            preferred_element_type=jnp.float32)

    return pl.pallas_call(
        body,
        out_shape=jax.ShapeDtypeStruct((8, INPUT_DIM), jnp.float32),
    )(emb_interaction, W[3 * D:4 * D, :])


def _tc_pack_table(tab_t):
    n_rows = tab_t.shape[1]
    block_v = PACK_BLOCK
    q = block_v // 4
    grid = (pl.cdiv(n_rows, block_v),)

    def body(t_ref, e_ref, o_ref):
        i = pl.program_id(0)
        cols = (jax.lax.broadcasted_iota(jnp.int32, (D, block_v), 1)
                + i * block_v)
        x = jnp.where(cols < n_rows, t_ref[...], 0.0)
        acc = jax.lax.dot_general(
            x[:, 0:q], e_ref[0:D, :], (((0,), (0,)), ((), ())),
            preferred_element_type=jnp.float32)
        for j in range(1, 4):
            acc += jax.lax.dot_general(
                x[:, q * j:q * (j + 1)], e_ref[D * j:D * (j + 1), :],
                (((0,), (0,)), ((), ())),
                preferred_element_type=jnp.float32)
        o_ref[...] = acc

    expander = jnp.eye(4 * D, dtype=jnp.float32)

    n_pad = grid[0] * block_v
    return pl.pallas_call(
        body,
        grid=grid,
        in_specs=[pl.BlockSpec((D, block_v), lambda i: (0, i)),
                  pl.BlockSpec((4 * D, 4 * D), lambda i: (0, 0))],
        out_specs=pl.BlockSpec((q, 4 * D), lambda i: (i, 0)),
        out_shape=jax.ShapeDtypeStruct((n_pad // 4, 4 * D), jnp.float32),
    )(tab_t, expander)


def _tc_project_packed(gi_p, gu_p, gt_p, inter3, wt, p_int, b, block_p):
    n_pack = gi_p.shape[0]
    grid = (n_pack // block_p,)
    block_n = 4 * block_p

    def body(gi_ref, gu_ref, gt_ref, int_ref, w1, w2, w3, p_ref, b_ref,
             o_ref):
        dn = (((1,), (0,)), ((), ()))
        acc = jax.lax.dot_general(
            gi_ref[...].astype(jnp.bfloat16), w1[...], dn,
            preferred_element_type=jnp.float32)
        acc += jax.lax.dot_general(
            gu_ref[...].astype(jnp.bfloat16), w2[...], dn,
            preferred_element_type=jnp.float32)
        acc += jax.lax.dot_general(
            gt_ref[...].astype(jnp.bfloat16), w3[...], dn,
            preferred_element_type=jnp.float32)
        ids = int_ref[0, 0, :]
        onehot = (ids[:, None]
                  == jax.lax.broadcasted_iota(jnp.int32, (block_n, 8), 1)
                  ).astype(jnp.float32)
        intc = jax.lax.dot_general(onehot, p_ref[...], dn,
                                   preferred_element_type=jnp.float32)
        o_ref[...] = acc.reshape(block_n, INPUT_DIM) + intc + b_ref[...]

    in_block = pl.BlockSpec((block_p, 4 * D), lambda i: (i, 0))
    return pl.pallas_call(
        body,
        grid=grid,
        in_specs=[in_block, in_block, in_block,
                  pl.BlockSpec((1, 1, block_n), lambda i: (i, 0, 0)),
                  pl.BlockSpec((4 * D, 4 * INPUT_DIM), lambda i: (0, 0)),
                  pl.BlockSpec((4 * D, 4 * INPUT_DIM), lambda i: (0, 0)),
                  pl.BlockSpec((4 * D, 4 * INPUT_DIM), lambda i: (0, 0)),
                  pl.BlockSpec((8, INPUT_DIM), lambda i: (0, 0)),
                  pl.BlockSpec((1, INPUT_DIM), lambda i: (0, 0))],
        out_specs=pl.BlockSpec((block_n, INPUT_DIM), lambda i: (i, 0)),
        out_shape=jax.ShapeDtypeStruct((4 * n_pack, INPUT_DIM), jnp.float32),
    )(gi_p, gu_p, gt_p, inter3, *wt, p_int, b.reshape(1, INPUT_DIM))


def kernel(item, user, tag, interaction, emb_item, emb_user, emb_tag,
           emb_interaction, W, b):
    B, L = item.shape
    n_rows = B * L
    n_pack = n_rows // 4
    block_p = 1024

    bv, qv = PACK_BLOCK, PACK_BLOCK // 4

    def perm(v):
        return (v // bv) * bv + 4 * (v % qv) + (v % bv) // qv

    idx_item = perm(item.T.reshape(-1).astype(jnp.int32))
    idx_user = perm(user.T.reshape(-1).astype(jnp.int32))
    idx_tag = tag.T.reshape(-1).astype(jnp.int32)
    inter3 = interaction.T.reshape(-1).astype(jnp.int32).reshape(
        n_pack // block_p, 1, 4 * block_p)

    item_pk = _tc_pack_table(emb_item.T)
    user_pk = _tc_pack_table(emb_user.T)
    item_rm = item_pk.reshape(item_pk.shape[0] * 4, D)
    user_rm = user_pk.reshape(user_pk.shape[0] * 4, D)

    gather = _sc_gather(n_rows, 400)
    gi, gu, gt = gather(item_rm, user_rm, emb_tag,
                        idx_item, idx_user, idx_tag)
    gi_p = gi.reshape(n_pack, 4 * D)
    gu_p = gu.reshape(n_pack, 4 * D)
    gt_p = gt.reshape(n_pack, 4 * D)

    eye4 = jnp.eye(4, dtype=jnp.bfloat16)
    wt = [jnp.kron(eye4, W[t * D:(t + 1) * D, :].astype(jnp.bfloat16))
          for t in range(3)]
    p_int = _tc_pint(emb_interaction, W)

    out = _tc_project_packed(gi_p, gu_p, gt_p, inter3, wt, p_int, b,
                             block_p)
    return out.reshape(L, B, INPUT_DIM).transpose(1, 0, 2)


# speedup vs baseline: 9.0319x; 1.0097x over previous
import functools

import jax
import jax.numpy as jnp
from jax import lax
from jax.experimental import pallas as pl
from jax.experimental.pallas import tpu as pltpu
from jax.experimental.pallas import tpu_sc as plsc

D = 32
INPUT_DIM = 128
PACK_BLOCK = 16384


def _sc_gather(n_rows, chunk, n_tab):
    info = plsc.get_sparse_core_info()
    nc, ns = info.num_cores, info.num_subcores
    nw = nc * ns
    rows_per_w = n_rows // nw
    assert n_rows % nw == 0 and rows_per_w % (2 * chunk) == 0
    n_chunks = rows_per_w // chunk

    mesh = plsc.VectorSubcoreMesh(core_axis_name="c", subcore_axis_name="s")

    @functools.partial(
        pl.kernel,
        mesh=mesh,
        compiler_params=pltpu.CompilerParams(use_tc_tiling_on_sc=False),
        out_type=[jax.ShapeDtypeStruct((n_rows, D), jnp.float32)
                  for _ in range(n_tab)],
        scratch_types=(
            [pltpu.VMEM((rows_per_w,), jnp.int32) for _ in range(n_tab)]
            + [pltpu.VMEM((chunk, D), jnp.float32)
               for _ in range(2 * n_tab)]
            + [pltpu.SemaphoreType.DMA for _ in range(5)]
        ),
    )
    def k(*refs):
        wid = lax.axis_index("s") * nc + lax.axis_index("c")
        wbase = wid * rows_per_w
        tabs = refs[0:n_tab]
        idxs = refs[n_tab:2 * n_tab]
        outs = refs[2 * n_tab:3 * n_tab]
        idx_v = refs[3 * n_tab:4 * n_tab]
        rows_v = (refs[4 * n_tab:5 * n_tab], refs[5 * n_tab:6 * n_tab])
        sem_l, sem_g0, sem_g1, sem_w0, sem_w1 = refs[6 * n_tab:]
        sem_g = (sem_g0, sem_g1)
        sem_w = (sem_w0, sem_w1)

        for t in range(n_tab):
            pltpu.async_copy(idxs[t].at[pl.ds(wbase, rows_per_w)],
                             idx_v[t], sem_l)
        for t in range(n_tab):
            pltpu.make_async_copy(idxs[t].at[pl.ds(wbase, rows_per_w)],
                                  idx_v[t], sem_l).wait()

        def fire_gathers(c, p):
            off = c * chunk
            for t in range(n_tab):
                pltpu.async_copy(
                    tabs[t].at[idx_v[t].at[pl.ds(off, chunk)]],
                    rows_v[p][t], sem_g[p])

        def drain_gathers(p):
            for t in range(n_tab):
                pltpu.make_async_copy(
                    tabs[t].at[idx_v[t].at[pl.ds(0, chunk)]],
                    rows_v[p][t], sem_g[p]).wait()

        def fire_wb(c, p):
            off = wbase + c * chunk
            for t in range(n_tab):
                pltpu.async_copy(rows_v[p][t],
                                 outs[t].at[pl.ds(off, chunk)], sem_w[p])

        def drain_wb(p):
            for t in range(n_tab):
                pltpu.make_async_copy(
                    rows_v[p][t], outs[t].at[pl.ds(0, chunk)],
                    sem_w[p]).wait()

        def process(c, p, is_first_pair):
            if not is_first_pair:
                drain_wb(p)
            fire_gathers(c, p)
            if p == 1:
                drain_gathers(0)
                fire_wb(c - 1, 0)
            elif not is_first_pair:
                drain_gathers(1)
                fire_wb(c - 1, 1)

        process(0, 0, True)
        process(1, 1, True)

        def body(j, _):
            c = 2 * j
            process(c, 0, False)
            process(c + 1, 1, False)
            return ()

        lax.fori_loop(1, n_chunks // 2, body, ())

        drain_gathers(1)
        fire_wb(n_chunks - 1, 1)
        drain_wb(0)
        drain_wb(1)

    return k


def _tc_pint(emb_interaction, W):
    n_int = emb_interaction.shape[0]

    def body(e_ref, w_ref, o_ref):
        o_ref[...] = jnp.zeros_like(o_ref)
        o_ref[0:n_int, :] = jax.lax.dot_general(
            e_ref[...], w_ref[...], (((1,), (0,)), ((), ())),
            preferred_element_type=jnp.float32)

    return pl.pallas_call(
        body,
        out_shape=jax.ShapeDtypeStruct((8, INPUT_DIM), jnp.float32),
    )(emb_interaction, W[3 * D:4 * D, :])


def _tc_pack_table(tab_t):
    n_rows = tab_t.shape[1]
    block_v = PACK_BLOCK
    q = block_v // 4
    grid = (pl.cdiv(n_rows, block_v),)

    def body(t_ref, e_ref, o_ref):
        i = pl.program_id(0)
        cols = (jax.lax.broadcasted_iota(jnp.int32, (D, block_v), 1)
                + i * block_v)
        x = jnp.where(cols < n_rows, t_ref[...], 0.0)
        acc = jax.lax.dot_general(
            x[:, 0:q], e_ref[0:D, :], (((0,), (0,)), ((), ())),
            preferred_element_type=jnp.float32)
        for j in range(1, 4):
            acc += jax.lax.dot_general(
                x[:, q * j:q * (j + 1)], e_ref[D * j:D * (j + 1), :],
                (((0,), (0,)), ((), ())),
                preferred_element_type=jnp.float32)
        o_ref[...] = acc

    expander = jnp.eye(4 * D, dtype=jnp.float32)

    n_pad = grid[0] * block_v
    return pl.pallas_call(
        body,
        grid=grid,
        in_specs=[pl.BlockSpec((D, block_v), lambda i: (0, i)),
                  pl.BlockSpec((4 * D, 4 * D), lambda i: (0, 0))],
        out_specs=pl.BlockSpec((q, 4 * D), lambda i: (i, 0)),
        out_shape=jax.ShapeDtypeStruct((n_pad // 4, 4 * D), jnp.float32),
    )(tab_t, expander)


def _tc_project_packed(gi_p, gu_p, gt_p, inter3, wt, p_int, b, block_p):
    n_pack = gi_p.shape[0]
    grid = (n_pack // block_p,)
    block_n = 4 * block_p

    def body(gi_ref, gu_ref, gt_ref, int_ref, w1, w2, w3, p_ref, b_ref,
             o_ref):
        dn = (((1,), (0,)), ((), ()))
        acc = jax.lax.dot_general(
            gi_ref[...].astype(jnp.bfloat16), w1[...], dn,
            preferred_element_type=jnp.float32)
        acc += jax.lax.dot_general(
            gu_ref[...].astype(jnp.bfloat16), w2[...], dn,
            preferred_element_type=jnp.float32)
        acc += jax.lax.dot_general(
            gt_ref[...].astype(jnp.bfloat16), w3[...], dn,
            preferred_element_type=jnp.float32)
        ids = int_ref[0, 0, :]
        onehot = (ids[:, None]
                  == jax.lax.broadcasted_iota(jnp.int32, (block_n, 8), 1)
                  ).astype(jnp.float32)
        intc = jax.lax.dot_general(onehot, p_ref[...], dn,
                                   preferred_element_type=jnp.float32)
        o_ref[...] = acc.reshape(block_n, INPUT_DIM) + intc + b_ref[...]

    in_block = pl.BlockSpec((block_p, 4 * D), lambda i: (i, 0))
    return pl.pallas_call(
        body,
        grid=grid,
        in_specs=[in_block, in_block, in_block,
                  pl.BlockSpec((1, 1, block_n), lambda i: (i, 0, 0)),
                  pl.BlockSpec((4 * D, 4 * INPUT_DIM), lambda i: (0, 0)),
                  pl.BlockSpec((4 * D, 4 * INPUT_DIM), lambda i: (0, 0)),
                  pl.BlockSpec((4 * D, 4 * INPUT_DIM), lambda i: (0, 0)),
                  pl.BlockSpec((8, INPUT_DIM), lambda i: (0, 0)),
                  pl.BlockSpec((1, INPUT_DIM), lambda i: (0, 0))],
        out_specs=pl.BlockSpec((block_n, INPUT_DIM), lambda i: (i, 0)),
        out_shape=jax.ShapeDtypeStruct((4 * n_pack, INPUT_DIM), jnp.float32),
    )(gi_p, gu_p, gt_p, inter3, *wt, p_int, b.reshape(1, INPUT_DIM))


def kernel(item, user, tag, interaction, emb_item, emb_user, emb_tag,
           emb_interaction, W, b):
    B, L = item.shape
    n_rows = B * L
    n_pack = n_rows // 4
    block_p = 1024

    bv, qv = PACK_BLOCK, PACK_BLOCK // 4

    def perm(v):
        return (v // bv) * bv + 4 * (v % qv) + (v % bv) // qv

    idx_item = perm(item.T.reshape(-1).astype(jnp.int32))
    idx_user = perm(user.T.reshape(-1).astype(jnp.int32))
    idx_tag = tag.T.reshape(-1).astype(jnp.int32)
    inter3 = interaction.T.reshape(-1).astype(jnp.int32).reshape(
        n_pack // block_p, 1, 4 * block_p)

    item_pk = _tc_pack_table(emb_item.T)
    user_pk = _tc_pack_table(emb_user.T)
    item_rm = item_pk.reshape(item_pk.shape[0] * 4, D)
    user_rm = user_pk.reshape(user_pk.shape[0] * 4, D)

    gu, gt = _sc_gather(n_rows, 400, 2)(user_rm, emb_tag, idx_user, idx_tag)
    (gi,) = _sc_gather(n_rows, 400, 1)(item_rm, idx_item)
    gi_p = gi.reshape(n_pack, 4 * D)
    gu_p = gu.reshape(n_pack, 4 * D)
    gt_p = gt.reshape(n_pack, 4 * D)

    eye4 = jnp.eye(4, dtype=jnp.bfloat16)
    wt = [jnp.kron(eye4, W[t * D:(t + 1) * D, :].astype(jnp.bfloat16))
          for t in range(3)]
    p_int = _tc_pint(emb_interaction, W)

    out = _tc_project_packed(gi_p, gu_p, gt_p, inter3, wt, p_int, b,
                             block_p)
    return out.reshape(L, B, INPUT_DIM).transpose(1, 0, 2)
